# Initial kernel scaffold; baseline (speedup 1.0000x reference)
import jax
import jax.numpy as jnp
from jax.experimental import pallas as pl


def kernel(x, edge_index, W_fc, b_fc, W_conv, b_conv):
    raise NotImplementedError("write your pallas kernel here")


# speedup vs baseline: 3.6456x; 3.6456x over previous
import dataclasses
import functools

import jax
import jax.numpy as jnp
from jax import lax
from jax.experimental import pallas as pl
from jax.experimental.pallas import tpu as pltpu
from jax.experimental.pallas import tpu_sc as plsc

N = 10000
E = 160000
D = 256

NC = 2
NS = 16
HALF = N // NC

DEG_ROWS = 80
DEG_COLS = 64

NW = NC * NS
TROWS = 320
DUMMY_ROW = TROWS
ECHUNK = 2000
N_ECHUNKS = E // ECHUNK
GBLK = 64
STAGE = ECHUNK + 16 + GBLK

_mesh = plsc.VectorSubcoreMesh(core_axis_name="core", subcore_axis_name="subcore")

_sc_params = pltpu.CompilerParams()
if "needs_layout_passes" in pltpu.CompilerParams.__dataclass_fields__:
    _sc_params = dataclasses.replace(_sc_params, needs_layout_passes=False)


@functools.partial(
    pl.kernel,
    mesh=_mesh,
    compiler_params=_sc_params,
    out_type=jax.ShapeDtypeStruct((NC, NS, DEG_ROWS, DEG_COLS), jnp.float32),
    scratch_types=[
        pltpu.VMEM((E // NS,), jnp.int32),
        pltpu.VMEM((DEG_ROWS, DEG_COLS), jnp.float32),
        pltpu.SemaphoreType.DMA,
    ],
)
def _deg_kernel(col_hbm, out_hbm, col_v, hist_v, sem):
    c = lax.axis_index("core")
    s = lax.axis_index("subcore")
    per_tile = E // NS

    @pl.loop(0, DEG_ROWS)
    def _(r):
        @pl.loop(0, DEG_COLS // 16)
        def _(k):
            hist_v[r, pl.ds(k * 16, 16)] = jnp.zeros((16,), jnp.float32)

    pltpu.sync_copy(col_hbm.at[pl.ds(s * per_tile, per_tile)], col_v)
    base = c * HALF
    ones = jnp.ones((16,), jnp.float32)

    @pl.loop(0, per_tile // 16)
    def _(g):
        idx = col_v[pl.ds(g * 16, 16)]
        loc = idx - base
        ok = (loc >= 0) & (loc < HALF)
        loc0 = jnp.where(ok, loc, 0)
        ri = jnp.right_shift(loc0, 6)
        ci = jnp.bitwise_and(loc0, 63)
        plsc.addupdate_scatter(hist_v, [ri, ci], ones, mask=ok)

    pltpu.sync_copy(hist_v, out_hbm.at[c].at[s])


_RB = 400


def _dense_body(x_ref, wfc_ref, bfc_ref, wcv_ref, bcv_ref, degt_ref,
                x0b_ref, hp_ref, dinv_ref):
    xb = x_ref[...]
    fc = jnp.dot(xb, wfc_ref[...], preferred_element_type=jnp.float32)
    x0b_ref[...] = jnp.maximum(fc + bfc_ref[...], 0.0) + bcv_ref[...]
    hv = jnp.dot(xb, wcv_ref[...], preferred_element_type=jnp.float32)
    deg = jnp.sum(degt_ref[...], axis=1, keepdims=True)
    dinv = jnp.where(deg > 0, lax.rsqrt(deg), 0.0)
    dinv_ref[...] = dinv
    hp_ref[...] = hv * dinv


def _dense_kernel(x, W_fc, b_fc, W_conv, b_conv, degT):
    grid = (N // _RB,)
    return pl.pallas_call(
        _dense_body,
        grid=grid,
        in_specs=[
            pl.BlockSpec((_RB, D), lambda i: (i, 0)),
            pl.BlockSpec((D, D), lambda i: (0, 0)),
            pl.BlockSpec((1, D), lambda i: (0, 0)),
            pl.BlockSpec((D, D), lambda i: (0, 0)),
            pl.BlockSpec((1, D), lambda i: (0, 0)),
            pl.BlockSpec((_RB, NS), lambda i: (i, 0)),
        ],
        out_specs=[
            pl.BlockSpec((_RB, D), lambda i: (i, 0)),
            pl.BlockSpec((_RB, D), lambda i: (i, 0)),
            pl.BlockSpec((_RB, 1), lambda i: (i, 0)),
        ],
        out_shape=[
            jax.ShapeDtypeStruct((N, D), jnp.float32),
            jax.ShapeDtypeStruct((N, D), jnp.float32),
            jax.ShapeDtypeStruct((N, 1), jnp.float32),
        ],
    )(x, W_fc, b_fc.reshape(1, D), W_conv, b_conv.reshape(1, D), degT)


@functools.partial(
    pl.kernel,
    mesh=_mesh,
    compiler_params=_sc_params,
    out_type=jax.ShapeDtypeStruct((NW, TROWS, D), jnp.float32),
    scratch_types=[
        pltpu.VMEM((ECHUNK,), jnp.int32),
        pltpu.VMEM((ECHUNK,), jnp.int32),
        pltpu.VMEM((STAGE,), jnp.int32),
        pltpu.VMEM((STAGE,), jnp.int32),
        pltpu.VMEM((GBLK, D), jnp.float32),
        pltpu.VMEM((TROWS + 8, D), jnp.float32),
        pltpu.SemaphoreType.DMA,
    ],
)
def _scatter_kernel(hp_hbm, row_hbm, col_hbm, out_hbm, row_v, col_v,
                    csrc_v, cdst_v, rows_v, acc_v, sem):
    c = lax.axis_index("core")
    s = lax.axis_index("subcore")
    w = c * NS + s
    lo = w * TROWS
    lanes = lax.iota(jnp.int32, 16)
    zeros16 = jnp.zeros((16,), jnp.float32)

    @pl.loop(0, TROWS + 8)
    def _(r):
        @pl.loop(0, D // 16)
        def _(k):
            acc_v[r, pl.ds(k * 16, 16)] = zeros16

    @pl.loop(0, N_ECHUNKS)
    def _(g):
        e0 = g * ECHUNK
        pltpu.sync_copy(row_hbm.at[pl.ds(e0, ECHUNK)], row_v)
        pltpu.sync_copy(col_hbm.at[pl.ds(e0, ECHUNK)], col_v)

        def scan_body(v, fill):
            dst = col_v[pl.ds(v * 16, 16)]
            src = row_v[pl.ds(v * 16, 16)]
            loc = dst - lo
            own = (loc >= 0) & (loc < TROWS)
            plsc.store_compressed(csrc_v.at[pl.ds(fill, 16)], src, mask=own)
            plsc.store_compressed(cdst_v.at[pl.ds(fill, 16)], loc, mask=own)
            cnt = plsc.all_reduce_population_count(own)
            return fill + cnt[0]

        fill = lax.fori_loop(0, ECHUNK // 16, scan_body, jnp.int32(0))

        @pl.loop(0, GBLK // 16)
        def _(p):
            csrc_v[pl.ds(fill + p * 16, 16)] = jnp.bitwise_and(
                lanes + p * 16 + g, 8191)
            cdst_v[pl.ds(fill + p * 16, 16)] = DUMMY_ROW + jnp.bitwise_and(
                lanes, 7)

        nblocks = (fill + (GBLK - 1)) // GBLK

        @pl.loop(0, nblocks)
        def _(b):
            pltpu.async_copy(hp_hbm.at[csrc_v.at[pl.ds(b * GBLK, GBLK)]],
                             rows_v, sem).wait()

            @pl.loop(0, GBLK)
            def _(r):
                dst = cdst_v[pl.ds(b * GBLK + r, 16)][0]
                dvec = dst + 0 * lanes

                @pl.loop(0, D // 16)
                def _(k):
                    v = rows_v[r, pl.ds(k * 16, 16)]
                    plsc.addupdate_scatter(acc_v, [dvec, k * 16 + lanes], v)

    pltpu.sync_copy(acc_v.at[pl.ds(0, TROWS)], out_hbm.at[c * NS + s])


def _combine_body(x0b_ref, dinv_ref, acc_ref, out_ref):
    out_ref[...] = x0b_ref[...] + dinv_ref[...] * acc_ref[...]


def _combine_kernel(x0b, dinv2d, acc):
    grid = (N // _RB,)
    return pl.pallas_call(
        _combine_body,
        grid=grid,
        in_specs=[
            pl.BlockSpec((_RB, D), lambda i: (i, 0)),
            pl.BlockSpec((_RB, 1), lambda i: (i, 0)),
            pl.BlockSpec((_RB, D), lambda i: (i, 0)),
        ],
        out_specs=pl.BlockSpec((_RB, D), lambda i: (i, 0)),
        out_shape=jax.ShapeDtypeStruct((N, D), jnp.float32),
    )(x0b, dinv2d, acc)


def kernel(x, edge_index, W_fc, b_fc, W_conv, b_conv):
    row = edge_index[0]
    col = edge_index[1]

    hist = _deg_kernel(col)
    hf = hist.reshape(NC, NS, DEG_ROWS * DEG_COLS)
    degT = jnp.concatenate(
        [hf[0, :, :HALF].T, hf[1, :, :HALF].T], axis=0
    )

    x0b, hp, dinv2d = _dense_kernel(x, W_fc, b_fc, W_conv, b_conv, degT)

    acc_parts = _scatter_kernel(hp, row, col)
    acc = acc_parts.reshape(NW * TROWS, D)[:N]

    return _combine_kernel(x0b, dinv2d, acc)


# speedup vs baseline: 4.2667x; 1.1704x over previous
import dataclasses
import functools

import jax
import jax.numpy as jnp
from jax import lax
from jax.experimental import pallas as pl
from jax.experimental.pallas import tpu as pltpu
from jax.experimental.pallas import tpu_sc as plsc

N = 10000
E = 160000
D = 256

NC = 2
NS = 16
HALF = N // NC

DEG_ROWS = 80
DEG_COLS = 64

NW = NC * NS
TROWS = 320
DUMMY_ROW = TROWS
ECHUNK = 4000
N_ECHUNKS = E // ECHUNK
GBLK = 64
STAGE = ECHUNK + 16 + GBLK

_mesh = plsc.VectorSubcoreMesh(core_axis_name="core", subcore_axis_name="subcore")

_sc_params = pltpu.CompilerParams()
if "needs_layout_passes" in pltpu.CompilerParams.__dataclass_fields__:
    _sc_params = dataclasses.replace(_sc_params, needs_layout_passes=False)


@functools.partial(
    pl.kernel,
    mesh=_mesh,
    compiler_params=_sc_params,
    out_type=jax.ShapeDtypeStruct((NC, NS, DEG_ROWS, DEG_COLS), jnp.float32),
    scratch_types=[
        pltpu.VMEM((E // NS,), jnp.int32),
        pltpu.VMEM((DEG_ROWS, DEG_COLS), jnp.float32),
        pltpu.SemaphoreType.DMA,
    ],
)
def _deg_kernel(col_hbm, out_hbm, col_v, hist_v, sem):
    c = lax.axis_index("core")
    s = lax.axis_index("subcore")
    per_tile = E // NS

    @pl.loop(0, DEG_ROWS)
    def _(r):
        @pl.loop(0, DEG_COLS // 16)
        def _(k):
            hist_v[r, pl.ds(k * 16, 16)] = jnp.zeros((16,), jnp.float32)

    pltpu.sync_copy(col_hbm.at[pl.ds(s * per_tile, per_tile)], col_v)
    base = c * HALF
    ones = jnp.ones((16,), jnp.float32)

    @pl.loop(0, per_tile // 16)
    def _(g):
        idx = col_v[pl.ds(g * 16, 16)]
        loc = idx - base
        ok = (loc >= 0) & (loc < HALF)
        loc0 = jnp.where(ok, loc, 0)
        ri = jnp.right_shift(loc0, 6)
        ci = jnp.bitwise_and(loc0, 63)
        plsc.addupdate_scatter(hist_v, [ri, ci], ones, mask=ok)

    pltpu.sync_copy(hist_v, out_hbm.at[c].at[s])


_RB = 400


def _dense_body(x_ref, wfc_ref, bfc_ref, wcv_ref, bcv_ref, degt_ref,
                x0b_ref, hp_ref, dinv_ref):
    xb = x_ref[...]
    fc = jnp.dot(xb, wfc_ref[...], preferred_element_type=jnp.float32)
    x0b_ref[...] = jnp.maximum(fc + bfc_ref[...], 0.0) + bcv_ref[...]
    hv = jnp.dot(xb, wcv_ref[...], preferred_element_type=jnp.float32)
    deg = jnp.sum(degt_ref[...], axis=1, keepdims=True)
    dinv = jnp.where(deg > 0, lax.rsqrt(deg), 0.0)
    dinv_ref[...] = dinv
    hp_ref[...] = hv * dinv


def _dense_kernel(x, W_fc, b_fc, W_conv, b_conv, degT):
    grid = (N // _RB,)
    return pl.pallas_call(
        _dense_body,
        grid=grid,
        in_specs=[
            pl.BlockSpec((_RB, D), lambda i: (i, 0)),
            pl.BlockSpec((D, D), lambda i: (0, 0)),
            pl.BlockSpec((1, D), lambda i: (0, 0)),
            pl.BlockSpec((D, D), lambda i: (0, 0)),
            pl.BlockSpec((1, D), lambda i: (0, 0)),
            pl.BlockSpec((_RB, NS), lambda i: (i, 0)),
        ],
        out_specs=[
            pl.BlockSpec((_RB, D), lambda i: (i, 0)),
            pl.BlockSpec((_RB, D), lambda i: (i, 0)),
            pl.BlockSpec((_RB, 1), lambda i: (i, 0)),
        ],
        out_shape=[
            jax.ShapeDtypeStruct((N, D), jnp.float32),
            jax.ShapeDtypeStruct((N, D), jnp.float32),
            jax.ShapeDtypeStruct((N, 1), jnp.float32),
        ],
    )(x, W_fc, b_fc.reshape(1, D), W_conv, b_conv.reshape(1, D), degT)


@functools.partial(
    pl.kernel,
    mesh=_mesh,
    compiler_params=_sc_params,
    out_type=jax.ShapeDtypeStruct((NW, TROWS, D), jnp.float32),
    scratch_types=[
        pltpu.VMEM((ECHUNK,), jnp.int32),
        pltpu.VMEM((ECHUNK,), jnp.int32),
        pltpu.VMEM((STAGE,), jnp.int32),
        pltpu.VMEM((STAGE,), jnp.int32),
        pltpu.VMEM((GBLK, D), jnp.float32),
        pltpu.VMEM((TROWS + 8, D), jnp.float32),
        pltpu.SemaphoreType.DMA,
    ],
)
def _scatter_kernel(hp_hbm, row_hbm, col_hbm, out_hbm, row_v, col_v,
                    csrc_v, cdst_v, rows_v, acc_v, sem):
    c = lax.axis_index("core")
    s = lax.axis_index("subcore")
    w = c * NS + s
    lo = w * TROWS
    lanes = lax.iota(jnp.int32, 16)
    zeros16 = jnp.zeros((16,), jnp.float32)

    @pl.loop(0, TROWS + 8)
    def _(r):
        @pl.loop(0, D // 16)
        def _(k):
            acc_v[r, pl.ds(k * 16, 16)] = zeros16

    @pl.loop(0, N_ECHUNKS)
    def _(g):
        e0 = g * ECHUNK
        pltpu.sync_copy(row_hbm.at[pl.ds(e0, ECHUNK)], row_v)
        pltpu.sync_copy(col_hbm.at[pl.ds(e0, ECHUNK)], col_v)

        def scan_body(v, fill):
            dst = col_v[pl.ds(v * 16, 16)]
            src = row_v[pl.ds(v * 16, 16)]
            loc = dst - lo
            own = (loc >= 0) & (loc < TROWS)
            plsc.store_compressed(csrc_v.at[pl.ds(fill, 16)], src, mask=own)
            plsc.store_compressed(cdst_v.at[pl.ds(fill, 16)], loc, mask=own)
            cnt = plsc.all_reduce_population_count(own)
            return fill + cnt[0]

        fill = lax.fori_loop(0, ECHUNK // 16, scan_body, jnp.int32(0))

        @pl.loop(0, GBLK // 16)
        def _(p):
            csrc_v[pl.ds(fill + p * 16, 16)] = jnp.bitwise_and(
                lanes + p * 16 + g, 8191)
            cdst_v[pl.ds(fill + p * 16, 16)] = DUMMY_ROW + jnp.bitwise_and(
                lanes, 7)

        nblocks = (fill + (GBLK - 1)) // GBLK

        @pl.loop(0, nblocks)
        def _(b):
            pltpu.async_copy(hp_hbm.at[csrc_v.at[pl.ds(b * GBLK, GBLK)]],
                             rows_v, sem).wait()

            @pl.loop(0, GBLK)
            def _(r):
                dst = cdst_v[pl.ds(b * GBLK + r, 16)][0]
                dvec = dst + 0 * lanes

                @pl.loop(0, D // 16)
                def _(k):
                    v = rows_v[r, pl.ds(k * 16, 16)]
                    plsc.addupdate_scatter(acc_v, [dvec, k * 16 + lanes], v)

    pltpu.sync_copy(acc_v.at[pl.ds(0, TROWS)], out_hbm.at[c * NS + s])


def _combine_body(x0b_ref, dinv_ref, acc_ref, out_ref):
    out_ref[...] = x0b_ref[...] + dinv_ref[...] * acc_ref[...]


def _combine_kernel(x0b, dinv2d, acc):
    grid = (N // _RB,)
    return pl.pallas_call(
        _combine_body,
        grid=grid,
        in_specs=[
            pl.BlockSpec((_RB, D), lambda i: (i, 0)),
            pl.BlockSpec((_RB, 1), lambda i: (i, 0)),
            pl.BlockSpec((_RB, D), lambda i: (i, 0)),
        ],
        out_specs=pl.BlockSpec((_RB, D), lambda i: (i, 0)),
        out_shape=jax.ShapeDtypeStruct((N, D), jnp.float32),
    )(x0b, dinv2d, acc)


def kernel(x, edge_index, W_fc, b_fc, W_conv, b_conv):
    row = edge_index[0]
    col = edge_index[1]

    hist = _deg_kernel(col)
    hf = hist.reshape(NC, NS, DEG_ROWS * DEG_COLS)
    degT = jnp.concatenate(
        [hf[0, :, :HALF].T, hf[1, :, :HALF].T], axis=0
    )

    x0b, hp, dinv2d = _dense_kernel(x, W_fc, b_fc, W_conv, b_conv, degT)

    acc_parts = _scatter_kernel(hp, row, col)
    acc = acc_parts.reshape(NW * TROWS, D)[:N]

    return _combine_kernel(x0b, dinv2d, acc)


# speedup vs baseline: 4.8149x; 1.1285x over previous
import dataclasses
import functools

import jax
import jax.numpy as jnp
from jax import lax
from jax.experimental import pallas as pl
from jax.experimental.pallas import tpu as pltpu
from jax.experimental.pallas import tpu_sc as plsc

N = 10000
E = 160000
D = 256

NC = 2
NS = 16
HALF = N // NC

DEG_ROWS = 80
DEG_COLS = 64

NW = NC * NS
TROWS = 320
DUMMY_ROW = TROWS
ECHUNK = 4000
N_ECHUNKS = E // ECHUNK
GBLK = 64
STAGE = ECHUNK + 16 + GBLK

_mesh = plsc.VectorSubcoreMesh(core_axis_name="core", subcore_axis_name="subcore")

_sc_params = pltpu.CompilerParams()
if "needs_layout_passes" in pltpu.CompilerParams.__dataclass_fields__:
    _sc_params = dataclasses.replace(_sc_params, needs_layout_passes=False)


@functools.partial(
    pl.kernel,
    mesh=_mesh,
    compiler_params=_sc_params,
    out_type=jax.ShapeDtypeStruct((NC, NS, DEG_ROWS, DEG_COLS), jnp.float32),
    scratch_types=[
        pltpu.VMEM((E // NS,), jnp.int32),
        pltpu.VMEM((DEG_ROWS, DEG_COLS), jnp.float32),
        pltpu.SemaphoreType.DMA,
    ],
)
def _deg_kernel(col_hbm, out_hbm, col_v, hist_v, sem):
    c = lax.axis_index("core")
    s = lax.axis_index("subcore")
    per_tile = E // NS

    @pl.loop(0, DEG_ROWS)
    def _(r):
        @pl.loop(0, DEG_COLS // 16)
        def _(k):
            hist_v[r, pl.ds(k * 16, 16)] = jnp.zeros((16,), jnp.float32)

    pltpu.sync_copy(col_hbm.at[pl.ds(s * per_tile, per_tile)], col_v)
    base = c * HALF
    ones = jnp.ones((16,), jnp.float32)

    @pl.loop(0, per_tile // 16)
    def _(g):
        idx = col_v[pl.ds(g * 16, 16)]
        loc = idx - base
        ok = (loc >= 0) & (loc < HALF)
        loc0 = jnp.where(ok, loc, 0)
        ri = jnp.right_shift(loc0, 6)
        ci = jnp.bitwise_and(loc0, 63)
        plsc.addupdate_scatter(hist_v, [ri, ci], ones, mask=ok)

    pltpu.sync_copy(hist_v, out_hbm.at[c].at[s])


_RB = 400


def _dense_body(x_ref, wfc_ref, bfc_ref, wcv_ref, bcv_ref, degt_ref,
                x0b_ref, hp_ref, dinv_ref):
    xb = x_ref[...]
    fc = jnp.dot(xb, wfc_ref[...], preferred_element_type=jnp.float32)
    x0b_ref[...] = jnp.maximum(fc + bfc_ref[...], 0.0) + bcv_ref[...]
    hv = jnp.dot(xb, wcv_ref[...], preferred_element_type=jnp.float32)
    deg = jnp.sum(degt_ref[...], axis=1, keepdims=True)
    dinv = jnp.where(deg > 0, lax.rsqrt(deg), 0.0)
    dinv_ref[...] = dinv
    hp_ref[...] = hv * dinv


def _dense_kernel(x, W_fc, b_fc, W_conv, b_conv, degT):
    grid = (N // _RB,)
    return pl.pallas_call(
        _dense_body,
        grid=grid,
        in_specs=[
            pl.BlockSpec((_RB, D), lambda i: (i, 0)),
            pl.BlockSpec((D, D), lambda i: (0, 0)),
            pl.BlockSpec((1, D), lambda i: (0, 0)),
            pl.BlockSpec((D, D), lambda i: (0, 0)),
            pl.BlockSpec((1, D), lambda i: (0, 0)),
            pl.BlockSpec((_RB, NS), lambda i: (i, 0)),
        ],
        out_specs=[
            pl.BlockSpec((_RB, D), lambda i: (i, 0)),
            pl.BlockSpec((_RB, D), lambda i: (i, 0)),
            pl.BlockSpec((_RB, 1), lambda i: (i, 0)),
        ],
        out_shape=[
            jax.ShapeDtypeStruct((N, D), jnp.float32),
            jax.ShapeDtypeStruct((N, D), jnp.float32),
            jax.ShapeDtypeStruct((N, 1), jnp.float32),
        ],
    )(x, W_fc, b_fc.reshape(1, D), W_conv, b_conv.reshape(1, D), degT)


@functools.partial(
    pl.kernel,
    mesh=_mesh,
    compiler_params=_sc_params,
    out_type=jax.ShapeDtypeStruct((NW, TROWS, D), jnp.float32),
    scratch_types=[
        pltpu.VMEM((ECHUNK,), jnp.int32),
        pltpu.VMEM((ECHUNK,), jnp.int32),
        pltpu.VMEM((STAGE,), jnp.int32),
        pltpu.VMEM((STAGE,), jnp.int32),
        pltpu.VMEM((GBLK, D), jnp.float32),
        pltpu.VMEM((TROWS + 8, D), jnp.float32),
        pltpu.SemaphoreType.DMA,
    ],
)
def _scatter_kernel(hp_hbm, row_hbm, col_hbm, out_hbm, row_v, col_v,
                    csrc_v, cdst_v, rows_v, acc_v, sem):
    c = lax.axis_index("core")
    s = lax.axis_index("subcore")
    w = c * NS + s
    lo = w * TROWS
    lanes = lax.iota(jnp.int32, 16)
    zeros16 = jnp.zeros((16,), jnp.float32)

    @pl.loop(0, TROWS + 8)
    def _(r):
        @pl.loop(0, D // 16)
        def _(k):
            acc_v[r, pl.ds(k * 16, 16)] = zeros16

    @pl.loop(0, N_ECHUNKS)
    def _(g):
        e0 = g * ECHUNK
        pltpu.sync_copy(row_hbm.at[pl.ds(e0, ECHUNK)], row_v)
        pltpu.sync_copy(col_hbm.at[pl.ds(e0, ECHUNK)], col_v)

        def scan_body(v, fill):
            dst = col_v[pl.ds(v * 16, 16)]
            src = row_v[pl.ds(v * 16, 16)]
            loc = dst - lo
            own = (loc >= 0) & (loc < TROWS)
            plsc.store_compressed(csrc_v.at[pl.ds(fill, 16)], src, mask=own)
            plsc.store_compressed(cdst_v.at[pl.ds(fill, 16)], loc, mask=own)
            cnt = plsc.all_reduce_population_count(own)
            return fill + cnt[0]

        fill = lax.fori_loop(0, ECHUNK // 16, scan_body, jnp.int32(0))

        @pl.loop(0, GBLK // 16)
        def _(p):
            csrc_v[pl.ds(fill + p * 16, 16)] = jnp.bitwise_and(
                lanes + p * 16 + g, 8191)
            cdst_v[pl.ds(fill + p * 16, 16)] = DUMMY_ROW + jnp.bitwise_and(
                lanes, 7)

        nblocks = (fill + (GBLK - 1)) // GBLK

        @pl.loop(0, nblocks)
        def _(b):
            pltpu.async_copy(hp_hbm.at[csrc_v.at[pl.ds(b * GBLK, GBLK)]],
                             rows_v, sem).wait()
            rmax = jnp.minimum(GBLK, fill - b * GBLK)

            @pl.loop(0, rmax)
            def _(r):
                dst = cdst_v[pl.ds(b * GBLK + r, 16)][0]
                dvec = dst + 0 * lanes

                @pl.loop(0, D // 16)
                def _(k):
                    v = rows_v[r, pl.ds(k * 16, 16)]
                    plsc.addupdate_scatter(acc_v, [dvec, k * 16 + lanes], v)

    pltpu.sync_copy(acc_v.at[pl.ds(0, TROWS)], out_hbm.at[c * NS + s])


def _combine_body(x0b_ref, dinv_ref, acc_ref, out_ref):
    out_ref[...] = x0b_ref[...] + dinv_ref[...] * acc_ref[...]


def _combine_kernel(x0b, dinv2d, acc):
    grid = (N // _RB,)
    return pl.pallas_call(
        _combine_body,
        grid=grid,
        in_specs=[
            pl.BlockSpec((_RB, D), lambda i: (i, 0)),
            pl.BlockSpec((_RB, 1), lambda i: (i, 0)),
            pl.BlockSpec((_RB, D), lambda i: (i, 0)),
        ],
        out_specs=pl.BlockSpec((_RB, D), lambda i: (i, 0)),
        out_shape=jax.ShapeDtypeStruct((N, D), jnp.float32),
    )(x0b, dinv2d, acc)


def kernel(x, edge_index, W_fc, b_fc, W_conv, b_conv):
    row = edge_index[0]
    col = edge_index[1]

    hist = _deg_kernel(col)
    hf = hist.reshape(NC, NS, DEG_ROWS * DEG_COLS)
    degT = jnp.concatenate(
        [hf[0, :, :HALF].T, hf[1, :, :HALF].T], axis=0
    )

    x0b, hp, dinv2d = _dense_kernel(x, W_fc, b_fc, W_conv, b_conv, degT)

    acc_parts = _scatter_kernel(hp, row, col)
    acc = acc_parts.reshape(NW * TROWS, D)[:N]

    return _combine_kernel(x0b, dinv2d, acc)
